# baseline (speedup 1.0000x reference)
import functools

import jax
import jax.numpy as jnp
from jax import lax
from jax.experimental import pallas as pl
from jax.experimental.pallas import tpu as pltpu
from jax.experimental.pallas import tpu_sc as plsc

_B = 32
_N = 1_000_000

_W = 65536
_C = 512

_SC_S = 6 * _W
_SC_CH = 4096
_TC_B0 = _SC_S // _W

_K0 = 0
_K1 = 42

_ROT_A = (13, 15, 26, 6)
_ROT_B = (17, 29, 16, 24)


def _rotl(x, d):
    return (x << jnp.uint32(d)) | (x >> jnp.uint32(32 - d))


def _threefry_bits(x0, x1):
    ks0 = jnp.uint32(_K0)
    ks1 = jnp.uint32(_K1)
    ks2 = jnp.uint32(_K0 ^ _K1 ^ 0x1BD11BDA)
    ks = (ks0, ks1, ks2)
    rots = (_ROT_A, _ROT_B)
    x0 = x0 + ks0
    x1 = x1 + ks1
    for i in range(5):
        for r in rots[i % 2]:
            x0 = x0 + x1
            x1 = _rotl(x1, r)
            x1 = x0 ^ x1
        x0 = x0 + ks[(i + 1) % 3]
        x1 = x1 + ks[(i + 2) % 3] + jnp.uint32(i + 1)
    return x0 ^ x1


def _bits_to_uniform(bits):
    fbits = (bits >> jnp.uint32(9)) | jnp.uint32(0x3F800000)
    u01 = lax.bitcast_convert_type(fbits, jnp.float32) - jnp.float32(1.0)
    minval = jnp.float32(1e-7)
    maxval = jnp.float32(1.0 - 1e-7)
    return jnp.maximum(minval, u01 * (maxval - minval) + minval)


def _gumbel_tc(lin_u32):
    u = _bits_to_uniform(_threefry_bits(jnp.zeros_like(lin_u32), lin_u32))
    return -jnp.log(-jnp.log(u))


def _tc_kernel(logits_ref, idx_ref, max_ref, runm_ref, runc_ref):
    j = pl.program_id(0)
    n_steps = pl.num_programs(0)
    base = (j + _TC_B0) * _W
    col0 = lax.broadcasted_iota(jnp.int32, (_B, _C), 1)
    rowoff = lax.broadcasted_iota(jnp.int32, (_B, _C), 0) * _N

    def body(k, carry):
        runm, runc = carry
        cbase = base + k * _C
        col = col0 + cbase
        lin = (rowoff + col).astype(jnp.uint32)
        x = logits_ref[:, pl.ds(k * _C, _C)] + _gumbel_tc(lin)
        x = jnp.where(col < _N, x, -jnp.inf)
        better = x > runm
        runm = jnp.where(better, x, runm)
        runc = jnp.where(better, col, runc)
        return runm, runc

    @pl.when(j == 0)
    def _init():
        runm_ref[...] = jnp.full((_B, _C), -jnp.inf, jnp.float32)
        runc_ref[...] = jnp.zeros((_B, _C), jnp.int32)

    init = (runm_ref[...], runc_ref[...])
    runm, runc = lax.fori_loop(0, _W // _C, body, init)
    runm_ref[...] = runm
    runc_ref[...] = runc

    @pl.when(j == n_steps - 1)
    def _finish():
        m = jnp.max(runm, axis=1, keepdims=True)
        big = jnp.int32(0x7FFFFFFF)
        bidx = jnp.min(jnp.where(runm == m, runc, big), axis=1, keepdims=True)
        max_ref[...] = m
        idx_ref[...] = bidx


def _tc_call(logits):
    n_blocks = pl.cdiv(_N - _SC_S, _W)
    idx, m = pl.pallas_call(
        _tc_kernel,
        grid=(n_blocks,),
        in_specs=[pl.BlockSpec((_B, _W), lambda j: (0, j + _TC_B0))],
        out_specs=[
            pl.BlockSpec((_B, 1), lambda j: (0, 0)),
            pl.BlockSpec((_B, 1), lambda j: (0, 0)),
        ],
        out_shape=[
            jax.ShapeDtypeStruct((_B, 1), jnp.int32),
            jax.ShapeDtypeStruct((_B, 1), jnp.float32),
        ],
        scratch_shapes=[
            pltpu.VMEM((_B, _C), jnp.float32),
            pltpu.VMEM((_B, _C), jnp.int32),
        ],
    )(logits)
    return idx, m


_LN2_HI = 0.69314575
_LN2_LO = 1.42860677e-06


def _log_sc(x):
    xb = lax.bitcast_convert_type(x, jnp.int32)
    k = (xb >> 23) - 127
    m = lax.bitcast_convert_type(
        (xb & jnp.int32(0x7FFFFF)) | jnp.int32(0x3F800000), jnp.float32)
    big = m > jnp.float32(1.4142135)
    m = jnp.where(big, m * jnp.float32(0.5), m)
    k = (k + jnp.where(big, jnp.int32(1), jnp.int32(0))).astype(jnp.float32)
    s = (m - jnp.float32(1.0)) / (m + jnp.float32(1.0))
    s2 = s * s
    p = s2 * (jnp.float32(1 / 3) + s2 * (jnp.float32(1 / 5)
              + s2 * jnp.float32(1 / 7)))
    t = jnp.float32(2.0) * (s + s * p)
    return k * jnp.float32(_LN2_HI) + (k * jnp.float32(_LN2_LO) + t)


def _gumbel_sc(lin_u32):
    u = _bits_to_uniform(_threefry_bits(jnp.zeros_like(lin_u32), lin_u32))
    return -_log_sc(-_log_sc(u))


def _sc_body(flat_hbm, outm_hbm, outc_hbm, buf, runm_ref, runc_ref):
    c = lax.axis_index("c")
    s = lax.axis_index("s")
    row = s * 2 + c
    lane = lax.iota(jnp.int32, 16)
    rowoff = row * _N
    flatbase = row * _SC_S

    def chunk_body(t, carry):
        cbase = t * _SC_CH
        pltpu.sync_copy(flat_hbm.at[pl.ds(flatbase + cbase, _SC_CH)], buf)

        def vec_body(v, carry2):
            runm, runc = carry2
            col = (cbase + v * 16) + lane
            lin = (rowoff + col).astype(jnp.uint32)
            x = buf[pl.ds(v * 16, 16)] + _gumbel_sc(lin)
            better = x > runm
            runm = jnp.where(better, x, runm)
            runc = jnp.where(better, col, runc)
            return runm, runc

        return lax.fori_loop(0, _SC_CH // 16, vec_body, carry)

    runm0 = jnp.full((16,), -jnp.inf, jnp.float32)
    runc0 = jnp.zeros((16,), jnp.int32)
    runm, runc = lax.fori_loop(0, _SC_S // _SC_CH, chunk_body, (runm0, runc0))
    runm_ref[...] = runm
    runc_ref[...] = runc
    pltpu.sync_copy(runm_ref, outm_hbm.at[row])
    pltpu.sync_copy(runc_ref, outc_hbm.at[row])


def _sc_call(flat):
    mesh = plsc.VectorSubcoreMesh(core_axis_name="c", subcore_axis_name="s")
    fn = pl.kernel(
        _sc_body,
        out_type=[
            jax.ShapeDtypeStruct((_B, 16), jnp.float32),
            jax.ShapeDtypeStruct((_B, 16), jnp.int32),
        ],
        mesh=mesh,
        scratch_types=[
            pltpu.MemorySpace.VMEM((_SC_CH,), jnp.float32),
            pltpu.MemorySpace.VMEM((16,), jnp.float32),
            pltpu.MemorySpace.VMEM((16,), jnp.int32),
        ],
        compiler_params=pltpu.CompilerParams(use_tc_tiling_on_sc=False),
    )
    return fn(flat)


def kernel(logits):
    tc_idx, tc_m = _tc_call(logits)
    flat = jnp.reshape(lax.slice(logits, (0, 0), (_B, _SC_S)), (_B * _SC_S,))
    sc_m, sc_c = _sc_call(flat)
    big = jnp.int32(0x7FFFFFFF)
    scm = jnp.max(sc_m, axis=1)
    scc = jnp.min(jnp.where(sc_m == scm[:, None], sc_c, big), axis=1)
    tcm = tc_m[:, 0]
    tcc = tc_idx[:, 0]
    return jnp.where(scm > tcm, scc, tcc).astype(jnp.int32)


# speedup vs baseline: 1.8508x; 1.8508x over previous
import functools

import jax
import jax.numpy as jnp
from jax import lax
from jax.experimental import pallas as pl
from jax.experimental.pallas import tpu as pltpu
from jax.experimental.pallas import tpu_sc as plsc

_B = 32
_N = 1_000_000

_W = 32768
_C = 512

_SC_S = 6 * _W
_SC_CH = 4096
_TC_B0 = _SC_S // _W

_K0 = 0
_K1 = 42

_ROT_A = (13, 15, 26, 6)
_ROT_B = (17, 29, 16, 24)


def _rotl(x, d):
    return (x << jnp.uint32(d)) | (x >> jnp.uint32(32 - d))


def _threefry_bits(x0, x1):
    ks0 = jnp.uint32(_K0)
    ks1 = jnp.uint32(_K1)
    ks2 = jnp.uint32(_K0 ^ _K1 ^ 0x1BD11BDA)
    ks = (ks0, ks1, ks2)
    rots = (_ROT_A, _ROT_B)
    x0 = x0 + ks0
    x1 = x1 + ks1
    for i in range(5):
        for r in rots[i % 2]:
            x0 = x0 + x1
            x1 = _rotl(x1, r)
            x1 = x0 ^ x1
        x0 = x0 + ks[(i + 1) % 3]
        x1 = x1 + ks[(i + 2) % 3] + jnp.uint32(i + 1)
    return x0 ^ x1


def _bits_to_uniform(bits):
    fbits = (bits >> jnp.uint32(9)) | jnp.uint32(0x3F800000)
    u01 = lax.bitcast_convert_type(fbits, jnp.float32) - jnp.float32(1.0)
    minval = jnp.float32(1e-7)
    maxval = jnp.float32(1.0 - 1e-7)
    return jnp.maximum(minval, u01 * (maxval - minval) + minval)


def _gumbel_tc(lin_u32):
    u = _bits_to_uniform(_threefry_bits(jnp.zeros_like(lin_u32), lin_u32))
    return -jnp.log(-jnp.log(u))


def _tc_kernel(logits_ref, idx_ref, max_ref, runm_ref, runc_ref):
    j = pl.program_id(0)
    n_steps = pl.num_programs(0)
    base = (j + _TC_B0) * _W
    col0 = lax.broadcasted_iota(jnp.int32, (_B, _C), 1)
    rowoff = lax.broadcasted_iota(jnp.int32, (_B, _C), 0) * _N

    def make_body(masked):
        def body(k, carry):
            runm, runc = carry
            cbase = base + k * _C
            col = col0 + cbase
            lin = (rowoff + col).astype(jnp.uint32)
            x = logits_ref[:, pl.ds(k * _C, _C)] + _gumbel_tc(lin)
            if masked:
                x = jnp.where(col < _N, x, -jnp.inf)
            better = x > runm
            runm = jnp.where(better, x, runm)
            runc = jnp.where(better, col, runc)
            return runm, runc
        return body

    @pl.when(j == 0)
    def _init():
        runm_ref[...] = jnp.full((_B, _C), -jnp.inf, jnp.float32)
        runc_ref[...] = jnp.zeros((_B, _C), jnp.int32)

    @pl.when(j < n_steps - 1)
    def _main():
        init = (runm_ref[...], runc_ref[...])
        runm, runc = lax.fori_loop(0, _W // _C, make_body(False), init)
        runm_ref[...] = runm
        runc_ref[...] = runc

    @pl.when(j == n_steps - 1)
    def _finish():
        init = (runm_ref[...], runc_ref[...])
        runm, runc = lax.fori_loop(0, _W // _C, make_body(True), init)
        m = jnp.max(runm, axis=1, keepdims=True)
        big = jnp.int32(0x7FFFFFFF)
        bidx = jnp.min(jnp.where(runm == m, runc, big), axis=1, keepdims=True)
        max_ref[...] = m
        idx_ref[...] = bidx


def _tc_call(logits):
    n_blocks = pl.cdiv(_N - _SC_S, _W)
    idx, m = pl.pallas_call(
        _tc_kernel,
        grid=(n_blocks,),
        in_specs=[pl.BlockSpec((_B, _W), lambda j: (0, j + _TC_B0))],
        out_specs=[
            pl.BlockSpec((_B, 1), lambda j: (0, 0)),
            pl.BlockSpec((_B, 1), lambda j: (0, 0)),
        ],
        out_shape=[
            jax.ShapeDtypeStruct((_B, 1), jnp.int32),
            jax.ShapeDtypeStruct((_B, 1), jnp.float32),
        ],
        scratch_shapes=[
            pltpu.VMEM((_B, _C), jnp.float32),
            pltpu.VMEM((_B, _C), jnp.int32),
        ],
    )(logits)
    return idx, m


_LN2_HI = 0.69314575
_LN2_LO = 1.42860677e-06


def _log_sc(x):
    xb = lax.bitcast_convert_type(x, jnp.int32)
    k = (xb >> 23) - 127
    m = lax.bitcast_convert_type(
        (xb & jnp.int32(0x7FFFFF)) | jnp.int32(0x3F800000), jnp.float32)
    big = m > jnp.float32(1.4142135)
    m = jnp.where(big, m * jnp.float32(0.5), m)
    k = (k + jnp.where(big, jnp.int32(1), jnp.int32(0))).astype(jnp.float32)
    s = (m - jnp.float32(1.0)) / (m + jnp.float32(1.0))
    s2 = s * s
    p = s2 * (jnp.float32(1 / 3) + s2 * (jnp.float32(1 / 5)
              + s2 * jnp.float32(1 / 7)))
    t = jnp.float32(2.0) * (s + s * p)
    return k * jnp.float32(_LN2_HI) + (k * jnp.float32(_LN2_LO) + t)


def _gumbel_sc(lin_u32):
    u = _bits_to_uniform(_threefry_bits(jnp.zeros_like(lin_u32), lin_u32))
    return -_log_sc(-_log_sc(u))


def _sc_body(flat_hbm, outm_hbm, outc_hbm, buf, runm_ref, runc_ref):
    c = lax.axis_index("c")
    s = lax.axis_index("s")
    row = s * 2 + c
    lane = lax.iota(jnp.int32, 16)
    rowoff = row * _N
    flatbase = row * _SC_S

    def chunk_body(t, carry):
        cbase = t * _SC_CH
        pltpu.sync_copy(flat_hbm.at[pl.ds(flatbase + cbase, _SC_CH)], buf)

        def vec_body(v, carry2):
            runm, runc = carry2
            col = (cbase + v * 16) + lane
            lin = (rowoff + col).astype(jnp.uint32)
            x = buf[pl.ds(v * 16, 16)] + _gumbel_sc(lin)
            better = x > runm
            runm = jnp.where(better, x, runm)
            runc = jnp.where(better, col, runc)
            return runm, runc

        return lax.fori_loop(0, _SC_CH // 16, vec_body, carry)

    runm0 = jnp.full((16,), -jnp.inf, jnp.float32)
    runc0 = jnp.zeros((16,), jnp.int32)
    runm, runc = lax.fori_loop(0, _SC_S // _SC_CH, chunk_body, (runm0, runc0))
    runm_ref[...] = runm
    runc_ref[...] = runc
    pltpu.sync_copy(runm_ref, outm_hbm.at[row])
    pltpu.sync_copy(runc_ref, outc_hbm.at[row])


def _sc_call(flat):
    mesh = plsc.VectorSubcoreMesh(core_axis_name="c", subcore_axis_name="s")
    fn = pl.kernel(
        _sc_body,
        out_type=[
            jax.ShapeDtypeStruct((_B, 16), jnp.float32),
            jax.ShapeDtypeStruct((_B, 16), jnp.int32),
        ],
        mesh=mesh,
        scratch_types=[
            pltpu.MemorySpace.VMEM((_SC_CH,), jnp.float32),
            pltpu.MemorySpace.VMEM((16,), jnp.float32),
            pltpu.MemorySpace.VMEM((16,), jnp.int32),
        ],
        compiler_params=pltpu.CompilerParams(use_tc_tiling_on_sc=False),
    )
    return fn(flat)


def kernel(logits):
    tc_idx, tc_m = _tc_call(logits)
    flat = jnp.reshape(lax.slice(logits, (0, 0), (_B, _SC_S)), (_B * _SC_S,))
    sc_m, sc_c = _sc_call(flat)
    big = jnp.int32(0x7FFFFFFF)
    scm = jnp.max(sc_m, axis=1)
    scc = jnp.min(jnp.where(sc_m == scm[:, None], sc_c, big), axis=1)
    tcm = tc_m[:, 0]
    tcc = tc_idx[:, 0]
    return jnp.where(scm > tcm, scc, tcc).astype(jnp.int32)


# speedup vs baseline: 1.8509x; 1.0001x over previous
import functools

import jax
import jax.numpy as jnp
from jax import lax
from jax.experimental import pallas as pl
from jax.experimental.pallas import tpu as pltpu
from jax.experimental.pallas import tpu_sc as plsc

_B = 32
_N = 1_000_000

_W = 32768
_C = 512

_SC_S = 6 * _W
_SC_CH = 4096
_TC_B0 = _SC_S // _W

_K0 = 0
_K1 = 42

_ROT_A = (13, 15, 26, 6)
_ROT_B = (17, 29, 16, 24)


def _rotl(x, d):
    return (x << jnp.uint32(d)) | (x >> jnp.uint32(32 - d))


def _threefry_bits(x0, x1):
    ks0 = jnp.uint32(_K0)
    ks1 = jnp.uint32(_K1)
    ks2 = jnp.uint32(_K0 ^ _K1 ^ 0x1BD11BDA)
    ks = (ks0, ks1, ks2)
    rots = (_ROT_A, _ROT_B)
    x0 = x0 + ks0
    x1 = x1 + ks1
    for i in range(5):
        for r in rots[i % 2]:
            x0 = x0 + x1
            x1 = _rotl(x1, r)
            x1 = x0 ^ x1
        x0 = x0 + ks[(i + 1) % 3]
        x1 = x1 + ks[(i + 2) % 3] + jnp.uint32(i + 1)
    return x0 ^ x1


def _bits_to_uniform(bits):
    fbits = (bits >> jnp.uint32(9)) | jnp.uint32(0x3F800000)
    u01 = lax.bitcast_convert_type(fbits, jnp.float32) - jnp.float32(1.0)
    minval = jnp.float32(1e-7)
    maxval = jnp.float32(1.0 - 1e-7)
    return jnp.maximum(minval, u01 * (maxval - minval) + minval)


def _gumbel_tc(lin_u32):
    u = _bits_to_uniform(_threefry_bits(jnp.zeros_like(lin_u32), lin_u32))
    return -jnp.log(-jnp.log(u))


def _tc_kernel(logits_ref, idx_ref, max_ref, runm_ref, runc_ref):
    j = pl.program_id(0)
    n_steps = pl.num_programs(0)
    base = (j + _TC_B0) * _W
    col0 = lax.broadcasted_iota(jnp.int32, (_B, _C), 1)
    rowoff = lax.broadcasted_iota(jnp.int32, (_B, _C), 0) * _N

    def make_body(masked):
        def body(k, carry):
            runm, runc = carry
            cbase = base + k * _C
            col = col0 + cbase
            lin = (rowoff + col).astype(jnp.uint32)
            x = logits_ref[:, pl.ds(k * _C, _C)] + _gumbel_tc(lin)
            if masked:
                x = jnp.where(col < _N, x, -jnp.inf)
            better = x > runm
            runm = jnp.where(better, x, runm)
            runc = jnp.where(better, col, runc)
            return runm, runc
        return body

    @pl.when(j == 0)
    def _init():
        runm_ref[...] = jnp.full((_B, _C), -jnp.inf, jnp.float32)
        runc_ref[...] = jnp.zeros((_B, _C), jnp.int32)

    @pl.when(j < n_steps - 1)
    def _main():
        init = (runm_ref[...], runc_ref[...])
        runm, runc = lax.fori_loop(0, _W // _C, make_body(False), init)
        runm_ref[...] = runm
        runc_ref[...] = runc

    @pl.when(j == n_steps - 1)
    def _finish():
        init = (runm_ref[...], runc_ref[...])
        runm, runc = lax.fori_loop(0, _W // _C, make_body(True), init)
        m = jnp.max(runm, axis=1, keepdims=True)
        big = jnp.int32(0x7FFFFFFF)
        bidx = jnp.min(jnp.where(runm == m, runc, big), axis=1, keepdims=True)
        max_ref[...] = m
        idx_ref[...] = bidx


def _tc_call(logits):
    n_blocks = pl.cdiv(_N - _SC_S, _W)
    idx, m = pl.pallas_call(
        _tc_kernel,
        grid=(n_blocks,),
        in_specs=[pl.BlockSpec((_B, _W), lambda j: (0, j + _TC_B0))],
        out_specs=[
            pl.BlockSpec((_B, 1), lambda j: (0, 0)),
            pl.BlockSpec((_B, 1), lambda j: (0, 0)),
        ],
        out_shape=[
            jax.ShapeDtypeStruct((_B, 1), jnp.int32),
            jax.ShapeDtypeStruct((_B, 1), jnp.float32),
        ],
        scratch_shapes=[
            pltpu.VMEM((_B, _C), jnp.float32),
            pltpu.VMEM((_B, _C), jnp.int32),
        ],
    )(logits)
    return idx, m


_LN2_HI = 0.69314575
_LN2_LO = 1.42860677e-06


def _log_sc(x):
    xb = lax.bitcast_convert_type(x, jnp.int32)
    k = (xb >> 23) - 127
    m = lax.bitcast_convert_type(
        (xb & jnp.int32(0x7FFFFF)) | jnp.int32(0x3F800000), jnp.float32)
    big = m > jnp.float32(1.4142135)
    m = jnp.where(big, m * jnp.float32(0.5), m)
    k = (k + jnp.where(big, jnp.int32(1), jnp.int32(0))).astype(jnp.float32)
    s = (m - jnp.float32(1.0)) / (m + jnp.float32(1.0))
    s2 = s * s
    p = s2 * (jnp.float32(1 / 3) + s2 * (jnp.float32(1 / 5)
              + s2 * jnp.float32(1 / 7)))
    t = jnp.float32(2.0) * (s + s * p)
    return k * jnp.float32(_LN2_HI) + (k * jnp.float32(_LN2_LO) + t)


def _gumbel_sc(lin_u32):
    u = _bits_to_uniform(_threefry_bits(jnp.zeros_like(lin_u32), lin_u32))
    return -_log_sc(-_log_sc(u))


def _sc_body(flat_hbm, outm_hbm, outc_hbm, buf, runm_ref, runc_ref):
    c = lax.axis_index("c")
    s = lax.axis_index("s")
    row = s * 2 + c
    lane = lax.iota(jnp.int32, 16)
    rowoff = row * _N
    flatbase = row * _SC_S

    def chunk_body(t, carry):
        cbase = t * _SC_CH
        pltpu.sync_copy(flat_hbm.at[pl.ds(flatbase + cbase, _SC_CH)], buf)

        def vec_body(v, carry2):
            runm_a, runc_a, runm_b, runc_b = carry2
            col_a = (cbase + v * 32) + lane
            col_b = col_a + 16
            x_a = buf[pl.ds(v * 32, 16)] \
                + _gumbel_sc((rowoff + col_a).astype(jnp.uint32))
            x_b = buf[pl.ds(v * 32 + 16, 16)] \
                + _gumbel_sc((rowoff + col_b).astype(jnp.uint32))
            ba = x_a > runm_a
            runm_a = jnp.where(ba, x_a, runm_a)
            runc_a = jnp.where(ba, col_a, runc_a)
            bb = x_b > runm_b
            runm_b = jnp.where(bb, x_b, runm_b)
            runc_b = jnp.where(bb, col_b, runc_b)
            return runm_a, runc_a, runm_b, runc_b

        return lax.fori_loop(0, _SC_CH // 32, vec_body, carry)

    runm0 = jnp.full((16,), -jnp.inf, jnp.float32)
    runc0 = jnp.zeros((16,), jnp.int32)
    runm_a, runc_a, runm_b, runc_b = lax.fori_loop(
        0, _SC_S // _SC_CH, chunk_body, (runm0, runc0, runm0, runc0))
    bb = runm_b > runm_a
    runm_ref[...] = jnp.where(bb, runm_b, runm_a)
    runc_ref[...] = jnp.where(bb, runc_b, runc_a)
    pltpu.sync_copy(runm_ref, outm_hbm.at[row])
    pltpu.sync_copy(runc_ref, outc_hbm.at[row])


def _sc_call(flat):
    mesh = plsc.VectorSubcoreMesh(core_axis_name="c", subcore_axis_name="s")
    fn = pl.kernel(
        _sc_body,
        out_type=[
            jax.ShapeDtypeStruct((_B, 16), jnp.float32),
            jax.ShapeDtypeStruct((_B, 16), jnp.int32),
        ],
        mesh=mesh,
        scratch_types=[
            pltpu.MemorySpace.VMEM((_SC_CH,), jnp.float32),
            pltpu.MemorySpace.VMEM((16,), jnp.float32),
            pltpu.MemorySpace.VMEM((16,), jnp.int32),
        ],
        compiler_params=pltpu.CompilerParams(use_tc_tiling_on_sc=False),
    )
    return fn(flat)


def kernel(logits):
    flat = jnp.reshape(lax.slice(logits, (0, 0), (_B, _SC_S)), (_B * _SC_S,))
    sc_m, sc_c = _sc_call(flat)
    tc_idx, tc_m = _tc_call(logits)
    big = jnp.int32(0x7FFFFFFF)
    scm = jnp.max(sc_m, axis=1)
    scc = jnp.min(jnp.where(sc_m == scm[:, None], sc_c, big), axis=1)
    tcm = tc_m[:, 0]
    tcc = tc_idx[:, 0]
    return jnp.where(scm > tcm, scc, tcc).astype(jnp.int32)
